# baseline (speedup 1.0000x reference)
import jax
import jax.numpy as jnp
from jax import lax
from jax.experimental import pallas as pl
from jax.experimental.pallas import tpu as pltpu
from jax.experimental.pallas import tpu_sc as plsc

N_NODES = 10000
D = 128
NC = 2
NS = 16
BK = 128
FAST_CORE = 0
NBF = 118
NBS = 40
NBT = NBF + NBS
EHALF = NBF // 2
EPAD = NS * NBT + (NBF - NBS)
ROWS_PER_TILE = 632
NROWS = NS * ROWS_PER_TILE


def _sc_agg_body(h_hbm, e_hbm, out_hbm, agg_sh,
                 e_v, gbuf0, gbuf1, sem0, sem1):
    gbufs = (gbuf0, gbuf1)
    gsems = (sem0, sem1)
    c = lax.axis_index("c")
    s = lax.axis_index("s")
    is_fast = c == FAST_CORE
    nb = lax.select(is_fast, jnp.int32(NBF), jnp.int32(NBS))
    off = lax.select(is_fast, jnp.int32(0), jnp.int32(NBF))
    eb0 = s * NBT + off

    zv = jnp.zeros((16,), jnp.float32)

    @pl.loop(0, BK)
    def _zrow(r):
        for k in range(D // 16):
            gbuf0[r, pl.ds(16 * k, 16)] = zv

    row0 = s * ROWS_PER_TILE
    nfull = ROWS_PER_TILE // BK
    for q in range(nfull):
        pltpu.sync_copy(gbuf0, agg_sh.at[pl.ds(row0 + q * BK, BK)])
    rrem = ROWS_PER_TILE - nfull * BK
    if rrem:
        pltpu.sync_copy(gbuf0.at[pl.ds(0, rrem)],
                        agg_sh.at[pl.ds(row0 + nfull * BK, rrem)])

    plsc.subcore_barrier()

    @pl.loop(0, 2)
    def _phase(p):
        pltpu.sync_copy(e_hbm.at[pl.ds(eb0 + p * EHALF, EHALF)], e_v)
        pn = lax.min(nb - p * EHALF, jnp.int32(EHALF))

        @pl.when(pn > 0)
        def _prime():
            pltpu.async_copy(h_hbm.at[e_v.at[0, 0]], gbufs[0], gsems[0])

        @pl.loop(0, (EHALF + 1) // 2)
        def _pair(t):
            for u in range(2):
                i = 2 * t + u

                @pl.when(i < pn)
                def _slot():
                    pltpu.make_async_copy(
                        h_hbm.at[e_v.at[0, 0]], gbufs[u], gsems[u]).wait()

                    @pl.when(i + 1 < pn)
                    def _prefetch():
                        pltpu.async_copy(h_hbm.at[e_v.at[i + 1, 0]],
                                         gbufs[1 - u], gsems[1 - u])

                    pltpu.sync_copy(gbufs[u], agg_sh.at[e_v.at[i, 1]],
                                    add=True)

    plsc.subcore_barrier()

    pltpu.sync_copy(agg_sh.at[pl.ds(row0, ROWS_PER_TILE)],
                    out_hbm.at[c, pl.ds(row0, ROWS_PER_TILE)])


@jax.jit
def _sc_agg(h, e_r):
    mesh = plsc.VectorSubcoreMesh(core_axis_name="c", subcore_axis_name="s")
    return pl.kernel(
        _sc_agg_body,
        out_type=jax.ShapeDtypeStruct((NC, NROWS, D), jnp.float32),
        mesh=mesh,
        scratch_types=[
            pltpu.VMEM_SHARED((NROWS, D), jnp.float32),
            pltpu.VMEM((EHALF, 2, BK), jnp.int32),
            pltpu.VMEM((BK, D), jnp.float32),
            pltpu.VMEM((BK, D), jnp.float32),
            pltpu.SemaphoreType.DMA,
            pltpu.SemaphoreType.DMA,
        ],
    )(h, e_r)


def _tc_update_body(h_ref, p0_ref, p1_ref, w_ref, b_ref, o_ref):
    x = h_ref[...] + p0_ref[0] + p1_ref[0]
    o_ref[...] = (
        jnp.dot(x, w_ref[...], preferred_element_type=jnp.float32,
                precision=lax.Precision.HIGHEST)
        + b_ref[...]
    )


@jax.jit
def _tc_update(h, parts, W, b2d):
    rb = 2000
    grid = (N_NODES // rb,)
    return pl.pallas_call(
        _tc_update_body,
        grid=grid,
        in_specs=[
            pl.BlockSpec((rb, D), lambda i: (i, 0)),
            pl.BlockSpec((1, rb, D), lambda i: (0, i, 0)),
            pl.BlockSpec((1, rb, D), lambda i: (1, i, 0)),
            pl.BlockSpec((D, D), lambda i: (0, 0)),
            pl.BlockSpec((1, D), lambda i: (0, 0)),
        ],
        out_specs=pl.BlockSpec((rb, D), lambda i: (i, 0)),
        out_shape=jax.ShapeDtypeStruct((N_NODES, D), jnp.float32),
    )(h, parts, parts, W, b2d)


def kernel(h, edge_index, W0, b0, W1, b1, W2, b2, W3, b3):
    src = edge_index[0].astype(jnp.int32)
    dst = edge_index[1].astype(jnp.int32)
    n_edges = src.shape[0]
    pad = EPAD * BK - n_edges
    src_b = jnp.concatenate(
        [src, jnp.zeros((pad,), jnp.int32)]).reshape(EPAD, BK)
    dst_b = jnp.concatenate(
        [dst, jnp.full((pad,), N_NODES, jnp.int32)]).reshape(EPAD, BK)
    e_r = jnp.stack([src_b, dst_b], axis=1)

    params = [(W0, b0), (W1, b1), (W2, b2), (W3, b3)]
    for W, b in params:
        parts = _sc_agg(h, e_r)
        h = _tc_update(h, parts, W, b.reshape(1, D))
    return h
